# baseline (speedup 1.0000x reference)
import functools

import jax
import jax.numpy as jnp
from jax import lax
from jax.experimental import pallas as pl
from jax.experimental.pallas import tpu as pltpu
from jax.experimental.pallas import tpu_sc as plsc

F = 1024
M = 3
L = 16
CH = F // L
NC = 2
NS = 16
NW = NC * NS
NR = 32


def kernel(x, modes, gamma, beta):
    B = x.shape[0]
    rows_per_w = B // NW
    nblk = rows_per_w // NR
    modes = modes.astype(jnp.int32)

    mesh = plsc.VectorSubcoreMesh(core_axis_name="c", subcore_axis_name="s")

    @functools.partial(
        pl.kernel,
        mesh=mesh,
        out_type=jax.ShapeDtypeStruct((B, F), jnp.float32),
        scratch_types=[
            pltpu.VMEM((rows_per_w,), jnp.int32),
            pltpu.VMEM((M, F), jnp.float32),
            pltpu.VMEM((M, F), jnp.float32),
            pltpu.VMEM((NR, F), jnp.float32),
        ],
    )
    def run(x_hbm, modes_hbm, gamma_hbm, beta_hbm, out_hbm,
            modes_v, g_v, b_v, xb):
        wid = lax.axis_index("s") * NC + lax.axis_index("c")
        row0 = wid * rows_per_w
        pltpu.sync_copy(modes_hbm.at[pl.ds(row0, rows_per_w)], modes_v)
        pltpu.sync_copy(gamma_hbm, g_v)
        pltpu.sync_copy(beta_hbm, b_v)

        def block_body(blk, _):
            rbase = row0 + blk * NR
            pltpu.sync_copy(x_hbm.at[pl.ds(rbase, NR)], xb)

            def group_body(g16, _):
                mvec = modes_v[pl.ds(blk * NR + g16 * L, L)]
                for i in range(L):
                    row = g16 * L + i
                    m = mvec[i]

                    def ch_body(j, _, row=row, m=m):
                        sl = pl.ds(j * L, L)
                        xv = xb[row, sl]
                        gv = g_v[m, sl]
                        bv = b_v[m, sl]
                        xb[row, sl] = gv * xv + bv
                        return 0

                    lax.fori_loop(0, CH, ch_body, 0, unroll=4)
                return 0

            lax.fori_loop(0, NR // L, group_body, 0)
            pltpu.sync_copy(xb, out_hbm.at[pl.ds(rbase, NR)])
            return 0

        lax.fori_loop(0, nblk, block_body, 0)

    return run(x, modes, gamma, beta)
